# baseline (speedup 1.0000x reference)
import functools

import jax
import jax.numpy as jnp
from jax.experimental import pallas as pl
from jax.experimental.pallas import tpu as pltpu
from jax.experimental.pallas import tpu_sc as plsc

_SHAPE = (4, 128, 2, 200, 176)


def _sc_copy(x_hbm, o_hbm, buf):
    c = jax.lax.axis_index("c")
    s = jax.lax.axis_index("s")
    wid = c * 16 + s
    b = wid // 8
    j = wid % 8
    for t in range(16):
        r = j * 16 + t
        for k in range(2):
            pltpu.sync_copy(x_hbm.at[b, r, k], buf)
            pltpu.sync_copy(buf, o_hbm.at[b, r, k])


def kernel(inputs):
    run = functools.partial(
        pl.kernel,
        out_type=jax.ShapeDtypeStruct(_SHAPE, jnp.float32),
        mesh=plsc.VectorSubcoreMesh(core_axis_name="c", subcore_axis_name="s"),
        scratch_types=[pltpu.VMEM((200, 176), jnp.float32)],
    )(_sc_copy)
    return run(inputs)


# speedup vs baseline: 1.0230x; 1.0230x over previous
import functools

import jax
import jax.numpy as jnp
from jax.experimental import pallas as pl
from jax.experimental.pallas import tpu as pltpu
from jax.experimental.pallas import tpu_sc as plsc

_SHAPE = (4, 128, 2, 200, 176)
_N_SLICES = 32


def _sc_copy(x_hbm, o_hbm, buf, in0, in1, out0, out1):
    c = jax.lax.axis_index("c")
    s = jax.lax.axis_index("s")
    wid = c * 16 + s
    b = wid // 8
    j = wid % 8
    in_sems = (in0, in1)
    out_sems = (out0, out1)

    def src(t):
        return x_hbm.at[b, j * 16 + t // 2, t % 2]

    def dst(t):
        return o_hbm.at[b, j * 16 + t // 2, t % 2]

    pltpu.make_async_copy(src(0), buf.at[0], in_sems[0]).start()
    for t in range(_N_SLICES):
        sl = t % 2
        pltpu.make_async_copy(src(t), buf.at[sl], in_sems[sl]).wait()
        if t >= 1:
            pltpu.make_async_copy(buf.at[1 - sl], dst(t - 1), out_sems[1 - sl]).wait()
        if t + 1 < _N_SLICES:
            pltpu.make_async_copy(src(t + 1), buf.at[1 - sl], in_sems[1 - sl]).start()
        pltpu.make_async_copy(buf.at[sl], dst(t), out_sems[sl]).start()
    last = (_N_SLICES - 1) % 2
    pltpu.make_async_copy(buf.at[last], dst(_N_SLICES - 1), out_sems[last]).wait()


def kernel(inputs):
    run = functools.partial(
        pl.kernel,
        out_type=jax.ShapeDtypeStruct(_SHAPE, jnp.float32),
        mesh=plsc.VectorSubcoreMesh(core_axis_name="c", subcore_axis_name="s"),
        scratch_types=[
            pltpu.VMEM((2, 200, 176), jnp.float32),
            pltpu.SemaphoreType.DMA,
            pltpu.SemaphoreType.DMA,
            pltpu.SemaphoreType.DMA,
            pltpu.SemaphoreType.DMA,
        ],
    )(_sc_copy)
    return run(inputs)


# speedup vs baseline: 1.3675x; 1.3368x over previous
import jax
import jax.numpy as jnp
from jax.experimental import pallas as pl
from jax.experimental.pallas import tpu as pltpu

_LEAD = 1024
_BLOCK = 32


def _copy_body(x_ref, o_ref):
    o_ref[...] = x_ref[...]


def kernel(inputs):
    x = inputs.reshape(_LEAD, 200, 176)
    out = pl.pallas_call(
        _copy_body,
        out_shape=jax.ShapeDtypeStruct(x.shape, x.dtype),
        grid=(_LEAD // _BLOCK,),
        in_specs=[pl.BlockSpec((_BLOCK, 200, 176), lambda i: (i, 0, 0))],
        out_specs=pl.BlockSpec((_BLOCK, 200, 176), lambda i: (i, 0, 0)),
    )(x)
    return out.reshape(inputs.shape)
